# baseline (speedup 1.0000x reference)
import jax
import jax.numpy as jnp
from jax.experimental import pallas as pl
from jax.experimental.pallas import tpu as pltpu

B, N, D, E, T, K = 2, 2048, 768, 8, 16, 2
NE = E + 1
BLK = 512
DCH = 256
NEG_INF = float("-inf")


def _moe_kernel(ids_ref, tt_ref, wg_ref, bg_ref, x_ref, w_ref, b_ref, out_ref):
    s_id = pl.program_id(0)
    xb = x_ref[...]

    tid = ids_ref[s_id // (N // BLK)]
    rows = jax.lax.broadcasted_iota(jnp.int32, (T, 1), 0)
    tvec = jnp.sum(jnp.where(rows == tid, tt_ref[...], 0.0), axis=0,
                   keepdims=True).astype(jnp.bfloat16)
    logits = (jnp.dot(xb, wg_ref[:D, :], preferred_element_type=jnp.float32)
              + jnp.dot(tvec, wg_ref[D:, :], preferred_element_type=jnp.float32)
              + bg_ref[...])
    lane = jax.lax.broadcasted_iota(jnp.int32, (BLK, E), 1)
    m1 = jnp.max(logits, axis=1, keepdims=True)
    idx1 = jnp.min(jnp.where(logits == m1, lane, E), axis=1, keepdims=True)
    masked = jnp.where(lane == idx1, NEG_INF, logits)
    m2 = jnp.max(masked, axis=1, keepdims=True)
    idx2 = jnp.min(jnp.where(masked == m2, lane, E), axis=1, keepdims=True)
    e2 = jnp.exp(m2 - m1)
    inv_s = 1.0 / (1.0 + e2)
    p1 = inv_s
    p2 = e2 * inv_s
    gates = (jnp.where(lane == idx1, p1, 0.0)
             + jnp.where(lane == idx2, p2, 0.0))
    omega = 1.0 - p1

    gcols = [gates[:, e:e + 1] for e in range(E)] + [omega]
    for dc in range(D // DCH):
        lo = dc * DCH
        acc = jnp.zeros((BLK, DCH), jnp.float32)
        for e in range(NE):
            h = (jnp.dot(xb, w_ref[e, :, lo:lo + DCH],
                         preferred_element_type=jnp.float32)
                 + b_ref[e, :, lo:lo + DCH])
            g = 0.5 * h * (1.0 + jax.lax.erf(h * 0.7071067811865476))
            acc = acc + gcols[e] * g
        out_ref[:, lo:lo + DCH] = acc


@jax.jit
def kernel(tokens, task_ids, task_table, Wg, bg, We, be, Wu, bu):
    x = tokens.reshape(B * N, D).astype(jnp.bfloat16)
    w_all = jnp.concatenate([We, Wu[None]], axis=0).astype(jnp.bfloat16)
    b_all = jnp.concatenate([be, bu[None]], axis=0)[:, None, :]
    wg_bf = Wg.astype(jnp.bfloat16)
    bg2 = bg.reshape(1, E)
    ids = task_ids.astype(jnp.int32)

    grid_spec = pltpu.PrefetchScalarGridSpec(
        num_scalar_prefetch=1,
        grid=(B * N // BLK,),
        in_specs=[
            pl.BlockSpec((T, D), lambda s, ids: (0, 0)),
            pl.BlockSpec((2 * D, E), lambda s, ids: (0, 0)),
            pl.BlockSpec((1, E), lambda s, ids: (0, 0)),
            pl.BlockSpec((BLK, D), lambda s, ids: (s, 0)),
            pl.BlockSpec((NE, D, D), lambda s, ids: (0, 0, 0)),
            pl.BlockSpec((NE, 1, D), lambda s, ids: (0, 0, 0)),
        ],
        out_specs=pl.BlockSpec((BLK, D), lambda s, ids: (s, 0)),
    )
    out = pl.pallas_call(
        _moe_kernel,
        grid_spec=grid_spec,
        out_shape=jax.ShapeDtypeStruct((B * N, D), jnp.float32),
        compiler_params=pltpu.CompilerParams(
            dimension_semantics=("arbitrary",),
        ),
    )(ids, task_table, wg_bf, bg2, x, w_all, b_all)
    return out.reshape(B, N, D)


# speedup vs baseline: 1.1302x; 1.1302x over previous
import jax
import jax.numpy as jnp
from jax.experimental import pallas as pl
from jax.experimental.pallas import tpu as pltpu

B, N, D, E, T, K = 2, 2048, 768, 8, 16, 2
NE = E + 1
BLK = 512
DCH = 256
NEG_INF = float("-inf")


def _moe_kernel(ids_ref, tt_ref, wg_ref, bg_ref, x_ref, we_ref, be_ref,
                wu_ref, bu_ref, out_ref):
    s_id = pl.program_id(0)
    xb = x_ref[...]

    tid = ids_ref[s_id // (N // BLK)]
    rows = jax.lax.broadcasted_iota(jnp.int32, (T, 1), 0)
    tvec = jnp.sum(jnp.where(rows == tid, tt_ref[...], 0.0), axis=0,
                   keepdims=True)
    logits = (jnp.dot(xb, wg_ref[:D, :], preferred_element_type=jnp.float32)
              + jnp.dot(tvec, wg_ref[D:, :], preferred_element_type=jnp.float32)
              + bg_ref[...])
    lane = jax.lax.broadcasted_iota(jnp.int32, (BLK, E), 1)
    m1 = jnp.max(logits, axis=1, keepdims=True)
    idx1 = jnp.min(jnp.where(logits == m1, lane, E), axis=1, keepdims=True)
    masked = jnp.where(lane == idx1, NEG_INF, logits)
    m2 = jnp.max(masked, axis=1, keepdims=True)
    idx2 = jnp.min(jnp.where(masked == m2, lane, E), axis=1, keepdims=True)
    e2 = jnp.exp(m2 - m1)
    inv_s = 1.0 / (1.0 + e2)
    p1 = inv_s
    p2 = e2 * inv_s
    gates = (jnp.where(lane == idx1, p1, 0.0)
             + jnp.where(lane == idx2, p2, 0.0))
    omega = 1.0 - p1

    gcols = [gates[:, e:e + 1] for e in range(E)] + [omega]
    for dc in range(D // DCH):
        lo = dc * DCH
        acc = jnp.zeros((BLK, DCH), jnp.float32)
        for e in range(NE):
            if e < E:
                w2d = we_ref[e, :, lo:lo + DCH]
                bias = be_ref[e, lo:lo + DCH]
            else:
                w2d = wu_ref[:, lo:lo + DCH]
                bias = bu_ref[0, lo:lo + DCH]
            h = jnp.dot(xb, w2d, preferred_element_type=jnp.float32) + bias
            g = 0.5 * h * (1.0 + jax.lax.erf(h * 0.7071067811865476))
            acc = acc + gcols[e] * g
        out_ref[:, lo:lo + DCH] = acc


@jax.jit
def kernel(tokens, task_ids, task_table, Wg, bg, We, be, Wu, bu):
    x = tokens.reshape(B * N, D)
    bg2 = bg.reshape(1, E)
    bu2 = bu.reshape(1, D)
    ids = task_ids.astype(jnp.int32)

    grid_spec = pltpu.PrefetchScalarGridSpec(
        num_scalar_prefetch=1,
        grid=(B * N // BLK,),
        in_specs=[
            pl.BlockSpec((T, D), lambda s, ids: (0, 0)),
            pl.BlockSpec((2 * D, E), lambda s, ids: (0, 0)),
            pl.BlockSpec((1, E), lambda s, ids: (0, 0)),
            pl.BlockSpec((BLK, D), lambda s, ids: (s, 0)),
            pl.BlockSpec((E, D, D), lambda s, ids: (0, 0, 0)),
            pl.BlockSpec((E, D), lambda s, ids: (0, 0)),
            pl.BlockSpec((D, D), lambda s, ids: (0, 0)),
            pl.BlockSpec((1, D), lambda s, ids: (0, 0)),
        ],
        out_specs=pl.BlockSpec((BLK, D), lambda s, ids: (s, 0)),
    )
    out = pl.pallas_call(
        _moe_kernel,
        grid_spec=grid_spec,
        out_shape=jax.ShapeDtypeStruct((B * N, D), jnp.float32),
        compiler_params=pltpu.CompilerParams(
            dimension_semantics=("arbitrary",),
        ),
    )(ids, task_table, Wg, bg2, x, We, be, Wu, bu2)
    return out.reshape(B, N, D)


# speedup vs baseline: 1.2328x; 1.0908x over previous
import jax
import jax.numpy as jnp
from jax.experimental import pallas as pl
from jax.experimental.pallas import tpu as pltpu

B, N, D, E, T, K = 2, 2048, 768, 8, 16, 2
NE = E + 1
BLK = 512
DCH = 256
NEG_INF = float("-inf")


def _moe_kernel(ids_ref, tt_ref, wg_ref, bg_ref, x_ref, we_ref, be_ref,
                wu_ref, bu_ref, out_ref, wbf_ref):
    s_id = pl.program_id(0)
    xb = x_ref[...]
    xbf = xb.astype(jnp.bfloat16)

    @pl.when(s_id == 0)
    def _stage_weights():
        wbf_ref[0:E] = we_ref[...].astype(jnp.bfloat16)
        wbf_ref[E] = wu_ref[...].astype(jnp.bfloat16)

    tid = ids_ref[s_id // (N // BLK)]
    rows = jax.lax.broadcasted_iota(jnp.int32, (T, 1), 0)
    tvec = jnp.sum(jnp.where(rows == tid, tt_ref[...], 0.0), axis=0,
                   keepdims=True)
    logits = (jnp.dot(xb, wg_ref[:D, :], preferred_element_type=jnp.float32)
              + jnp.dot(tvec, wg_ref[D:, :], preferred_element_type=jnp.float32)
              + bg_ref[...])
    lane = jax.lax.broadcasted_iota(jnp.int32, (BLK, E), 1)
    m1 = jnp.max(logits, axis=1, keepdims=True)
    idx1 = jnp.min(jnp.where(logits == m1, lane, E), axis=1, keepdims=True)
    masked = jnp.where(lane == idx1, NEG_INF, logits)
    m2 = jnp.max(masked, axis=1, keepdims=True)
    idx2 = jnp.min(jnp.where(masked == m2, lane, E), axis=1, keepdims=True)
    e2 = jnp.exp(m2 - m1)
    inv_s = 1.0 / (1.0 + e2)
    p1 = inv_s
    p2 = e2 * inv_s
    gates = (jnp.where(lane == idx1, p1, 0.0)
             + jnp.where(lane == idx2, p2, 0.0))
    omega = 1.0 - p1

    gcols = [gates[:, e:e + 1] for e in range(E)] + [omega]
    for dc in range(D // DCH):
        lo = dc * DCH
        acc = jnp.zeros((BLK, DCH), jnp.float32)
        for e in range(NE):
            w2d = wbf_ref[e, :, lo:lo + DCH]
            bias = (be_ref[e, lo:lo + DCH] if e < E
                    else bu_ref[0, lo:lo + DCH])
            h = jnp.dot(xbf, w2d, preferred_element_type=jnp.float32) + bias
            g = 0.5 * h * (1.0 + jax.lax.erf(h * 0.7071067811865476))
            acc = acc + gcols[e] * g
        out_ref[:, lo:lo + DCH] = acc


@jax.jit
def kernel(tokens, task_ids, task_table, Wg, bg, We, be, Wu, bu):
    x = tokens.reshape(B * N, D)
    bg2 = bg.reshape(1, E)
    bu2 = bu.reshape(1, D)
    ids = task_ids.astype(jnp.int32)

    grid_spec = pltpu.PrefetchScalarGridSpec(
        num_scalar_prefetch=1,
        grid=(B * N // BLK,),
        in_specs=[
            pl.BlockSpec((T, D), lambda s, ids: (0, 0)),
            pl.BlockSpec((2 * D, E), lambda s, ids: (0, 0)),
            pl.BlockSpec((1, E), lambda s, ids: (0, 0)),
            pl.BlockSpec((BLK, D), lambda s, ids: (s, 0)),
            pl.BlockSpec((E, D, D), lambda s, ids: (0, 0, 0)),
            pl.BlockSpec((E, D), lambda s, ids: (0, 0)),
            pl.BlockSpec((D, D), lambda s, ids: (0, 0)),
            pl.BlockSpec((1, D), lambda s, ids: (0, 0)),
        ],
        out_specs=pl.BlockSpec((BLK, D), lambda s, ids: (s, 0)),
        scratch_shapes=[pltpu.VMEM((NE, D, D), jnp.bfloat16)],
    )
    out = pl.pallas_call(
        _moe_kernel,
        grid_spec=grid_spec,
        out_shape=jax.ShapeDtypeStruct((B * N, D), jnp.float32),
        compiler_params=pltpu.CompilerParams(
            dimension_semantics=("arbitrary",),
        ),
    )(ids, task_table, Wg, bg2, x, We, be, Wu, bu2)
    return out.reshape(B, N, D)


# speedup vs baseline: 1.2665x; 1.0273x over previous
import jax
import jax.numpy as jnp
from jax.experimental import pallas as pl
from jax.experimental.pallas import tpu as pltpu

B, N, D, E, T, K = 2, 2048, 768, 8, 16, 2
NE = E + 1
BLK = 1024
DCH = 256
NEG_INF = float("-inf")


def _moe_kernel(ids_ref, tt_ref, wg_ref, bg_ref, x_ref, we_ref, be_ref,
                wu_ref, bu_ref, out_ref, wbf_ref):
    s_id = pl.program_id(0)
    xb = x_ref[...]
    xbf = xb.astype(jnp.bfloat16)

    @pl.when(s_id == 0)
    def _stage_weights():
        wbf_ref[0:E] = we_ref[...].astype(jnp.bfloat16)
        wbf_ref[E] = wu_ref[...].astype(jnp.bfloat16)

    tid = ids_ref[s_id // (N // BLK)]
    rows = jax.lax.broadcasted_iota(jnp.int32, (T, 1), 0)
    tvec = jnp.sum(jnp.where(rows == tid, tt_ref[...], 0.0), axis=0,
                   keepdims=True)
    logits = (jnp.dot(xb, wg_ref[:D, :], preferred_element_type=jnp.float32)
              + jnp.dot(tvec, wg_ref[D:, :], preferred_element_type=jnp.float32)
              + bg_ref[...])
    lane = jax.lax.broadcasted_iota(jnp.int32, (BLK, E), 1)
    m1 = jnp.max(logits, axis=1, keepdims=True)
    idx1 = jnp.min(jnp.where(logits == m1, lane, E), axis=1, keepdims=True)
    masked = jnp.where(lane == idx1, NEG_INF, logits)
    m2 = jnp.max(masked, axis=1, keepdims=True)
    idx2 = jnp.min(jnp.where(masked == m2, lane, E), axis=1, keepdims=True)
    e2 = jnp.exp(m2 - m1)
    inv_s = 1.0 / (1.0 + e2)
    p1 = inv_s
    p2 = e2 * inv_s
    gates = (jnp.where(lane == idx1, p1, 0.0)
             + jnp.where(lane == idx2, p2, 0.0))
    omega = 1.0 - p1

    gcols = [gates[:, e:e + 1] for e in range(E)] + [omega]
    for dc in range(D // DCH):
        lo = dc * DCH
        acc = jnp.zeros((BLK, DCH), jnp.float32)
        for e in range(NE):
            w2d = wbf_ref[e, :, lo:lo + DCH]
            bias = (be_ref[e, lo:lo + DCH] if e < E
                    else bu_ref[0, lo:lo + DCH])
            h = jnp.dot(xbf, w2d, preferred_element_type=jnp.float32) + bias
            g = 0.5 * h * (1.0 + jax.lax.erf(h * 0.7071067811865476))
            acc = acc + gcols[e] * g
        out_ref[:, lo:lo + DCH] = acc


@jax.jit
def kernel(tokens, task_ids, task_table, Wg, bg, We, be, Wu, bu):
    x = tokens.reshape(B * N, D)
    bg2 = bg.reshape(1, E)
    bu2 = bu.reshape(1, D)
    ids = task_ids.astype(jnp.int32)

    grid_spec = pltpu.PrefetchScalarGridSpec(
        num_scalar_prefetch=1,
        grid=(B * N // BLK,),
        in_specs=[
            pl.BlockSpec((T, D), lambda s, ids: (0, 0)),
            pl.BlockSpec((2 * D, E), lambda s, ids: (0, 0)),
            pl.BlockSpec((1, E), lambda s, ids: (0, 0)),
            pl.BlockSpec((BLK, D), lambda s, ids: (s, 0)),
            pl.BlockSpec((E, D, D), lambda s, ids: (0, 0, 0)),
            pl.BlockSpec((E, D), lambda s, ids: (0, 0)),
            pl.BlockSpec((D, D), lambda s, ids: (0, 0)),
            pl.BlockSpec((1, D), lambda s, ids: (0, 0)),
        ],
        out_specs=pl.BlockSpec((BLK, D), lambda s, ids: (s, 0)),
        scratch_shapes=[pltpu.VMEM((NE, D, D), jnp.bfloat16)],
    )
    out = pl.pallas_call(
        _moe_kernel,
        grid_spec=grid_spec,
        out_shape=jax.ShapeDtypeStruct((B * N, D), jnp.float32),
        compiler_params=pltpu.CompilerParams(
            dimension_semantics=("arbitrary",),
        ),
    )(ids, task_table, Wg, bg2, x, We, be, Wu, bu2)
    return out.reshape(B, N, D)
